# Initial kernel scaffold; baseline (speedup 1.0000x reference)
import jax
import jax.numpy as jnp
from jax.experimental import pallas as pl


def kernel(x, edge_index, W_self1, W_nbr1, b1, W_self2, W_nbr2, b2, Wp1, bp1, Wp2, bp2):
    raise NotImplementedError("write your pallas kernel here")


# speedup vs baseline: 5.5193x; 5.5193x over previous
import jax
import jax.numpy as jnp
from jax import lax
from jax.experimental import pallas as pl
from jax.experimental.pallas import tpu as pltpu
from jax.experimental.pallas import tpu_sc as plsc

D = 128
DH = D // 2
NC = 2
NS = 16
CH = 80


def _sc_agg(E, N, want_deg):
    NB = E // CH
    NK = NB // NS
    NP = ((N + 255) // 256) * 256
    RPT = NP // NS
    SEG = NP // NS
    FL = 128
    mesh = plsc.VectorSubcoreMesh(core_axis_name="c", subcore_axis_name="s")

    out_type = [jax.ShapeDtypeStruct((NC, NP, DH), jnp.float32)]
    scratch = [
        pltpu.VMEM((NK, CH), jnp.int32),
        pltpu.VMEM((NK, CH), jnp.int32),
        pltpu.VMEM((CH, DH), jnp.float32),
        pltpu.VMEM((FL, DH), jnp.float32),
        pltpu.VMEM_SHARED((NP, DH), jnp.float32),
        pltpu.SemaphoreType.DMA,
    ]
    if want_deg:
        out_type.append(jax.ShapeDtypeStruct((NP,), jnp.float32))
        scratch += [
            pltpu.VMEM((NP,), jnp.float32),
            pltpu.VMEM((SEG,), jnp.float32),
            pltpu.VMEM_SHARED((NS, NP), jnp.float32),
        ]

    def body(src2d_h, dst2d_h, ga_h, gb_h, z_h, *rest):
        if want_deg:
            (part_h, deg_h, srcb, dstb, rows, zstg, agg_sh, sem,
             hist, tmp, dparts_sh) = rest
        else:
            part_h, srcb, dstb, rows, zstg, agg_sh, sem = rest
        c = lax.axis_index("c")
        s = lax.axis_index("s")

        pltpu.sync_copy(z_h, zstg)
        for q in range(RPT // FL):
            pltpu.sync_copy(zstg, agg_sh.at[pl.ds(s * RPT + q * FL, FL)])
        if want_deg:
            def z_body(i, carry):
                hist[pl.ds(i * 16, 16)] = jnp.zeros((16,), jnp.float32)
                return carry
            lax.fori_loop(0, NP // 16, z_body, 0)
        plsc.subcore_barrier()

        pltpu.sync_copy(src2d_h.at[pl.ds(s * NK, NK)], srcb)
        pltpu.sync_copy(dst2d_h.at[pl.ds(s * NK, NK)], dstb)

        if want_deg:
            ones = jnp.ones((16,), jnp.float32)

            def h_body(k, carry):
                for j in range(CH // 16):
                    v = dstb[k, pl.ds(j * 16, 16)]
                    plsc.addupdate_scatter(hist, [v], ones)
                return carry
            lax.fori_loop(0, NK, h_body, 0)

        def agg_loop(g_ref):
            def a_body(k, carry):
                pltpu.async_copy(g_ref.at[srcb.at[k]], rows, sem).wait()
                pltpu.sync_copy(rows, agg_sh.at[dstb.at[k]], add=True)
                return carry
            lax.fori_loop(0, NK, a_body, 0)

        @pl.when(c == 0)
        def _():
            agg_loop(ga_h)

        @pl.when(c == 1)
        def _():
            agg_loop(gb_h)

        if want_deg:
            pltpu.sync_copy(hist, dparts_sh.at[s])
        plsc.subcore_barrier()

        for q in range(RPT // FL):
            pltpu.sync_copy(agg_sh.at[pl.ds(s * RPT + q * FL, FL)], zstg)
            pltpu.sync_copy(zstg, part_h.at[c, pl.ds(s * RPT + q * FL, FL)])

        if want_deg:
            @pl.when(c == 0)
            def _():
                def r_body(i, carry):
                    tmp[pl.ds(i * 16, 16)] = jnp.zeros((16,), jnp.float32)
                    return carry
                lax.fori_loop(0, SEG // 16, r_body, 0)
                for t in range(NS):
                    pltpu.sync_copy(
                        dparts_sh.at[t, pl.ds(s * SEG, SEG)],
                        hist.at[pl.ds(0, SEG)])

                    def add_body(i, carry):
                        sl = pl.ds(i * 16, 16)
                        tmp[sl] = tmp[sl] + hist[sl]
                        return carry
                    lax.fori_loop(0, SEG // 16, add_body, 0)
                pltpu.sync_copy(tmp, deg_h.at[pl.ds(s * SEG, SEG)])

    ot = tuple(out_type) if want_deg else out_type[0]
    return pl.kernel(
        body, out_type=ot, mesh=mesh, scratch_types=scratch,
        compiler_params=pltpu.CompilerParams(
            use_tc_tiling_on_sc=False, needs_layout_passes=False),
        name="sc_seg_agg")


def _tc_in(x, Wn, Ws, b):
    N = x.shape[0]
    BN = 1000
    grid = (N // BN,)

    def body(xb, wn, ws, bb, g_out, s_out):
        xv = xb[...]
        g_out[...] = jnp.dot(xv, wn[...], preferred_element_type=jnp.float32)
        s_out[...] = jnp.dot(xv, ws[...], preferred_element_type=jnp.float32) + bb[...]

    return pl.pallas_call(
        body,
        grid=grid,
        in_specs=[
            pl.BlockSpec((BN, D), lambda i: (i, 0)),
            pl.BlockSpec((D, D), lambda i: (0, 0)),
            pl.BlockSpec((D, D), lambda i: (0, 0)),
            pl.BlockSpec((1, D), lambda i: (0, 0)),
        ],
        out_specs=[
            pl.BlockSpec((BN, D), lambda i: (i, 0)),
            pl.BlockSpec((BN, D), lambda i: (i, 0)),
        ],
        out_shape=[
            jax.ShapeDtypeStruct((N, D), jnp.float32),
            jax.ShapeDtypeStruct((N, D), jnp.float32),
        ],
        name="tc_in",
    )(x, Wn, Ws, b.reshape(1, D))


def _tc_mid(S, part, deg, Wn, Ws, b):
    N = S.shape[0]
    BN = 1000
    grid = (N // BN,)

    def body(sb, pa, pb, dg, wn, ws, bb, g_out, s_out):
        d = jnp.maximum(dg[...], 1.0)
        agg = jnp.concatenate([pa[0], pb[0]], axis=1)
        h = jnp.maximum(sb[...] + agg / d, 0.0)
        g_out[...] = jnp.dot(h, wn[...], preferred_element_type=jnp.float32)
        s_out[...] = jnp.dot(h, ws[...], preferred_element_type=jnp.float32) + bb[...]

    return pl.pallas_call(
        body,
        grid=grid,
        in_specs=[
            pl.BlockSpec((BN, D), lambda i: (i, 0)),
            pl.BlockSpec((1, BN, DH), lambda i: (0, i, 0)),
            pl.BlockSpec((1, BN, DH), lambda i: (1, i, 0)),
            pl.BlockSpec((BN, 1), lambda i: (i, 0)),
            pl.BlockSpec((D, D), lambda i: (0, 0)),
            pl.BlockSpec((D, D), lambda i: (0, 0)),
            pl.BlockSpec((1, D), lambda i: (0, 0)),
        ],
        out_specs=[
            pl.BlockSpec((BN, D), lambda i: (i, 0)),
            pl.BlockSpec((BN, D), lambda i: (i, 0)),
        ],
        out_shape=[
            jax.ShapeDtypeStruct((N, D), jnp.float32),
            jax.ShapeDtypeStruct((N, D), jnp.float32),
        ],
        name="tc_mid",
    )(S, part, part, deg, Wn, Ws, b.reshape(1, D))


def _tc_out(S, part, deg, Wp1, bp1, Wp2, bp2):
    N = S.shape[0]
    P_HID = Wp1.shape[1]
    P_OUT = Wp2.shape[1]
    BN = 1000
    grid = (N // BN,)

    def body(sb, pa, pb, dg, w1, b1b, w2, b2b, out):
        d = jnp.maximum(dg[...], 1.0)
        agg = jnp.concatenate([pa[0], pb[0]], axis=1)
        h = jnp.maximum(sb[...] + agg / d, 0.0)
        t = jnp.maximum(
            jnp.dot(h, w1[...], preferred_element_type=jnp.float32) + b1b[...], 0.0)
        out[...] = jnp.dot(t, w2[...], preferred_element_type=jnp.float32) + b2b[...]

    return pl.pallas_call(
        body,
        grid=grid,
        in_specs=[
            pl.BlockSpec((BN, D), lambda i: (i, 0)),
            pl.BlockSpec((1, BN, DH), lambda i: (0, i, 0)),
            pl.BlockSpec((1, BN, DH), lambda i: (1, i, 0)),
            pl.BlockSpec((BN, 1), lambda i: (i, 0)),
            pl.BlockSpec((D, P_HID), lambda i: (0, 0)),
            pl.BlockSpec((1, P_HID), lambda i: (0, 0)),
            pl.BlockSpec((P_HID, P_OUT), lambda i: (0, 0)),
            pl.BlockSpec((1, P_OUT), lambda i: (0, 0)),
        ],
        out_specs=pl.BlockSpec((BN, P_OUT), lambda i: (i, 0)),
        out_shape=jax.ShapeDtypeStruct((N, P_OUT), jnp.float32),
        name="tc_out",
    )(S, part, part, deg, Wp1, bp1.reshape(1, P_HID), Wp2, bp2.reshape(1, P_OUT))


def kernel(x, edge_index, W_self1, W_nbr1, b1, W_self2, W_nbr2, b2,
           Wp1, bp1, Wp2, bp2):
    N = x.shape[0]
    E = edge_index.shape[1]
    src2d = edge_index[0].reshape(E // CH, CH)
    dst2d = edge_index[1].reshape(E // CH, CH)
    zeros2d = jnp.zeros((128, DH), jnp.float32)

    agg_deg = _sc_agg(E, N, True)
    agg = _sc_agg(E, N, False)

    G1, S1 = _tc_in(x, W_nbr1, W_self1, b1)
    part1, deg = agg_deg(src2d, dst2d, G1[:, :DH], G1[:, DH:], zeros2d)
    degc = deg[:N].reshape(N, 1)
    G2, S2 = _tc_mid(S1, part1, degc, W_nbr2, W_self2, b2)
    part2 = agg(src2d, dst2d, G2[:, :DH], G2[:, DH:], zeros2d)
    return _tc_out(S2, part2, degc, Wp1, bp1, Wp2, bp2)


# speedup vs baseline: 8.8780x; 1.6085x over previous
import jax
import jax.numpy as jnp
from jax import lax
from jax.experimental import pallas as pl
from jax.experimental.pallas import tpu as pltpu
from jax.experimental.pallas import tpu_sc as plsc

D = 128
DH = D // 2
NC = 2
NS = 16
CH = 80


def _sc_agg(E, N, want_deg):
    NB = E // CH
    NK = NB // NS
    NP = ((N + 255) // 256) * 256
    RPT = NP // NS
    SEG = NP // NS
    FL = 128
    mesh = plsc.VectorSubcoreMesh(core_axis_name="c", subcore_axis_name="s")

    out_type = [jax.ShapeDtypeStruct((NC, NP, DH), jnp.float32)]
    scratch = [
        pltpu.VMEM((NK, CH), jnp.int32),
        pltpu.VMEM((NK, CH), jnp.int32),
        pltpu.VMEM((CH, DH), jnp.float32),
        pltpu.VMEM((CH, DH), jnp.float32),
        pltpu.VMEM((FL, DH), jnp.float32),
        pltpu.VMEM_SHARED((NP, DH), jnp.float32),
        pltpu.SemaphoreType.DMA,
        pltpu.SemaphoreType.DMA,
    ]
    if want_deg:
        out_type.append(jax.ShapeDtypeStruct((NP,), jnp.float32))
        scratch += [
            pltpu.VMEM((NP,), jnp.float32),
            pltpu.VMEM((SEG,), jnp.float32),
            pltpu.VMEM_SHARED((NS, NP), jnp.float32),
        ]

    def body(src2d_h, dst2d_h, ga_h, gb_h, z_h, *rest):
        if want_deg:
            (part_h, deg_h, srcb, dstb, rows0, rows1, zstg, agg_sh,
             sem0, sem1, hist, tmp, dparts_sh) = rest
        else:
            part_h, srcb, dstb, rows0, rows1, zstg, agg_sh, sem0, sem1 = rest
        c = lax.axis_index("c")
        s = lax.axis_index("s")

        pltpu.sync_copy(z_h, zstg)
        for q in range(RPT // FL):
            pltpu.sync_copy(zstg, agg_sh.at[pl.ds(s * RPT + q * FL, FL)])
        if want_deg:
            def z_body(i, carry):
                hist[pl.ds(i * 16, 16)] = jnp.zeros((16,), jnp.float32)
                return carry
            lax.fori_loop(0, NP // 16, z_body, 0)
        plsc.subcore_barrier()

        pltpu.sync_copy(src2d_h.at[pl.ds(s * NK, NK)], srcb)
        pltpu.sync_copy(dst2d_h.at[pl.ds(s * NK, NK)], dstb)

        ones = jnp.ones((16,), jnp.float32)

        def hist_chunk(k):
            if want_deg:
                for j in range(CH // 16):
                    v = dstb[k, pl.ds(j * 16, 16)]
                    plsc.addupdate_scatter(hist, [v], ones)

        def agg_loop(g_ref):
            pltpu.async_copy(g_ref.at[srcb.at[0]], rows0, sem0)

            def a_body(i, carry):
                k0 = 2 * i
                k1 = 2 * i + 1
                pltpu.async_copy(g_ref.at[srcb.at[k1]], rows1, sem1)
                hist_chunk(k0)
                pltpu.make_async_copy(g_ref.at[srcb.at[k0]], rows0, sem0).wait()
                pltpu.sync_copy(rows0, agg_sh.at[dstb.at[k0]], add=True)

                @pl.when(i < NK // 2 - 1)
                def _():
                    pltpu.async_copy(g_ref.at[srcb.at[k0 + 2]], rows0, sem0)
                hist_chunk(k1)
                pltpu.make_async_copy(g_ref.at[srcb.at[k1]], rows1, sem1).wait()
                pltpu.sync_copy(rows1, agg_sh.at[dstb.at[k1]], add=True)
                return carry
            lax.fori_loop(0, NK // 2, a_body, 0)

        @pl.when(c == 0)
        def _():
            agg_loop(ga_h)

        @pl.when(c == 1)
        def _():
            agg_loop(gb_h)

        if want_deg:
            pltpu.sync_copy(hist, dparts_sh.at[s])
        plsc.subcore_barrier()

        for q in range(RPT // FL):
            pltpu.sync_copy(agg_sh.at[pl.ds(s * RPT + q * FL, FL)], zstg)
            pltpu.sync_copy(zstg, part_h.at[c, pl.ds(s * RPT + q * FL, FL)])

        if want_deg:
            @pl.when(c == 0)
            def _():
                def r_body(i, carry):
                    tmp[pl.ds(i * 16, 16)] = jnp.zeros((16,), jnp.float32)
                    return carry
                lax.fori_loop(0, SEG // 16, r_body, 0)
                for t in range(NS):
                    pltpu.sync_copy(
                        dparts_sh.at[t, pl.ds(s * SEG, SEG)],
                        hist.at[pl.ds(0, SEG)])

                    def add_body(i, carry):
                        sl = pl.ds(i * 16, 16)
                        tmp[sl] = tmp[sl] + hist[sl]
                        return carry
                    lax.fori_loop(0, SEG // 16, add_body, 0)
                pltpu.sync_copy(tmp, deg_h.at[pl.ds(s * SEG, SEG)])

    ot = tuple(out_type) if want_deg else out_type[0]
    return pl.kernel(
        body, out_type=ot, mesh=mesh, scratch_types=scratch,
        compiler_params=pltpu.CompilerParams(
            use_tc_tiling_on_sc=False, needs_layout_passes=False),
        name="sc_seg_agg")


def _tc_in(x, Wn, Ws, b):
    N = x.shape[0]
    BN = 1000
    grid = (N // BN,)

    def body(xb, wn, ws, bb, g_out, s_out):
        xv = xb[...]
        g_out[...] = jnp.dot(xv, wn[...], preferred_element_type=jnp.float32)
        s_out[...] = jnp.dot(xv, ws[...], preferred_element_type=jnp.float32) + bb[...]

    return pl.pallas_call(
        body,
        grid=grid,
        in_specs=[
            pl.BlockSpec((BN, D), lambda i: (i, 0)),
            pl.BlockSpec((D, D), lambda i: (0, 0)),
            pl.BlockSpec((D, D), lambda i: (0, 0)),
            pl.BlockSpec((1, D), lambda i: (0, 0)),
        ],
        out_specs=[
            pl.BlockSpec((BN, D), lambda i: (i, 0)),
            pl.BlockSpec((BN, D), lambda i: (i, 0)),
        ],
        out_shape=[
            jax.ShapeDtypeStruct((N, D), jnp.float32),
            jax.ShapeDtypeStruct((N, D), jnp.float32),
        ],
        name="tc_in",
    )(x, Wn, Ws, b.reshape(1, D))


def _tc_mid(S, part, deg, Wn, Ws, b):
    N = S.shape[0]
    BN = 1000
    grid = (N // BN,)

    def body(sb, pa, pb, dg, wn, ws, bb, g_out, s_out):
        d = jnp.maximum(dg[...], 1.0)
        agg = jnp.concatenate([pa[0], pb[0]], axis=1)
        h = jnp.maximum(sb[...] + agg / d, 0.0)
        g_out[...] = jnp.dot(h, wn[...], preferred_element_type=jnp.float32)
        s_out[...] = jnp.dot(h, ws[...], preferred_element_type=jnp.float32) + bb[...]

    return pl.pallas_call(
        body,
        grid=grid,
        in_specs=[
            pl.BlockSpec((BN, D), lambda i: (i, 0)),
            pl.BlockSpec((1, BN, DH), lambda i: (0, i, 0)),
            pl.BlockSpec((1, BN, DH), lambda i: (1, i, 0)),
            pl.BlockSpec((BN, 1), lambda i: (i, 0)),
            pl.BlockSpec((D, D), lambda i: (0, 0)),
            pl.BlockSpec((D, D), lambda i: (0, 0)),
            pl.BlockSpec((1, D), lambda i: (0, 0)),
        ],
        out_specs=[
            pl.BlockSpec((BN, D), lambda i: (i, 0)),
            pl.BlockSpec((BN, D), lambda i: (i, 0)),
        ],
        out_shape=[
            jax.ShapeDtypeStruct((N, D), jnp.float32),
            jax.ShapeDtypeStruct((N, D), jnp.float32),
        ],
        name="tc_mid",
    )(S, part, part, deg, Wn, Ws, b.reshape(1, D))


def _tc_out(S, part, deg, Wp1, bp1, Wp2, bp2):
    N = S.shape[0]
    P_HID = Wp1.shape[1]
    P_OUT = Wp2.shape[1]
    BN = 1000
    grid = (N // BN,)

    def body(sb, pa, pb, dg, w1, b1b, w2, b2b, out):
        d = jnp.maximum(dg[...], 1.0)
        agg = jnp.concatenate([pa[0], pb[0]], axis=1)
        h = jnp.maximum(sb[...] + agg / d, 0.0)
        t = jnp.maximum(
            jnp.dot(h, w1[...], preferred_element_type=jnp.float32) + b1b[...], 0.0)
        out[...] = jnp.dot(t, w2[...], preferred_element_type=jnp.float32) + b2b[...]

    return pl.pallas_call(
        body,
        grid=grid,
        in_specs=[
            pl.BlockSpec((BN, D), lambda i: (i, 0)),
            pl.BlockSpec((1, BN, DH), lambda i: (0, i, 0)),
            pl.BlockSpec((1, BN, DH), lambda i: (1, i, 0)),
            pl.BlockSpec((BN, 1), lambda i: (i, 0)),
            pl.BlockSpec((D, P_HID), lambda i: (0, 0)),
            pl.BlockSpec((1, P_HID), lambda i: (0, 0)),
            pl.BlockSpec((P_HID, P_OUT), lambda i: (0, 0)),
            pl.BlockSpec((1, P_OUT), lambda i: (0, 0)),
        ],
        out_specs=pl.BlockSpec((BN, P_OUT), lambda i: (i, 0)),
        out_shape=jax.ShapeDtypeStruct((N, P_OUT), jnp.float32),
        name="tc_out",
    )(S, part, part, deg, Wp1, bp1.reshape(1, P_HID), Wp2, bp2.reshape(1, P_OUT))


def kernel(x, edge_index, W_self1, W_nbr1, b1, W_self2, W_nbr2, b2,
           Wp1, bp1, Wp2, bp2):
    N = x.shape[0]
    E = edge_index.shape[1]
    src2d = edge_index[0].reshape(E // CH, CH)
    dst2d = edge_index[1].reshape(E // CH, CH)
    zeros2d = jnp.zeros((128, DH), jnp.float32)

    agg_deg = _sc_agg(E, N, True)
    agg = _sc_agg(E, N, False)

    G1, S1 = _tc_in(x, W_nbr1, W_self1, b1)
    part1, deg = agg_deg(src2d, dst2d, G1[:, :DH], G1[:, DH:], zeros2d)
    degc = deg[:N].reshape(N, 1)
    G2, S2 = _tc_mid(S1, part1, degc, W_nbr2, W_self2, b2)
    part2 = agg(src2d, dst2d, G2[:, :DH], G2[:, DH:], zeros2d)
    return _tc_out(S2, part2, degc, Wp1, bp1, Wp2, bp2)


# speedup vs baseline: 11.1144x; 1.2519x over previous
import jax
import jax.numpy as jnp
from jax import lax
from jax.experimental import pallas as pl
from jax.experimental.pallas import tpu as pltpu
from jax.experimental.pallas import tpu_sc as plsc

D = 128
DH = D // 2
NC = 2
NS = 16
CH = 80


def _sc_agg(E, N, want_deg):
    NB = E // CH
    NK = NB // NS
    NP = ((N + 255) // 256) * 256
    RPT = NP // NS
    SEG = NP // NS
    FL = 128
    mesh = plsc.VectorSubcoreMesh(core_axis_name="c", subcore_axis_name="s")

    out_type = [jax.ShapeDtypeStruct((NC, NP, DH), jnp.float32)]
    scratch = [
        pltpu.VMEM((NK, CH), jnp.int32),
        pltpu.VMEM((NK, CH), jnp.int32),
        pltpu.VMEM((CH, DH), jnp.float32),
        pltpu.VMEM((CH, DH), jnp.float32),
        pltpu.VMEM((CH, DH), jnp.float32),
        pltpu.VMEM((CH, DH), jnp.float32),
        pltpu.VMEM((CH, DH), jnp.float32),
        pltpu.VMEM((FL, DH), jnp.float32),
        pltpu.VMEM_SHARED((NP, DH), jnp.float32),
    ] + [pltpu.SemaphoreType.DMA] * 10
    if want_deg:
        out_type.append(jax.ShapeDtypeStruct((NP,), jnp.float32))
        out_type.append(jax.ShapeDtypeStruct((NS, NP), jnp.float32))
        scratch += [
            pltpu.VMEM((NP,), jnp.float32),
            pltpu.VMEM((SEG,), jnp.float32),
        ]

    def body(src2d_h, dst2d_h, ga_h, gb_h, z_h, *rest):
        if want_deg:
            (part_h, deg_h, dparts_sh, srcb, dstb, r0, r1, r2, r3, r4,
             zstg, agg_sh,
             sg0, sg1, sg2, sg3, sg4, ss0, ss1, ss2, ss3, ss4,
             hist, tmp) = rest
        else:
            (part_h, srcb, dstb, r0, r1, r2, r3, r4, zstg, agg_sh,
             sg0, sg1, sg2, sg3, sg4, ss0, ss1, ss2, ss3, ss4) = rest
        rows = [r0, r1, r2, r3, r4]
        sg = [sg0, sg1, sg2, sg3, sg4]
        ss = [ss0, ss1, ss2, ss3, ss4]
        c = lax.axis_index("c")
        s = lax.axis_index("s")

        pltpu.sync_copy(z_h, zstg)
        for q in range(RPT // FL):
            pltpu.sync_copy(zstg, agg_sh.at[pl.ds(s * RPT + q * FL, FL)])
        if want_deg:
            def z_body(i, carry):
                hist[pl.ds(i * 16, 16)] = jnp.zeros((16,), jnp.float32)
                return carry
            lax.fori_loop(0, NP // 16, z_body, 0)
        plsc.subcore_barrier()

        pltpu.sync_copy(src2d_h.at[pl.ds(s * NK, NK)], srcb)
        pltpu.sync_copy(dst2d_h.at[pl.ds(s * NK, NK)], dstb)

        ones = jnp.ones((16,), jnp.float32)

        def hist_chunk(k):
            if want_deg:
                for j in range(CH // 16):
                    v = dstb[k, pl.ds(j * 16, 16)]
                    plsc.addupdate_scatter(hist, [v], ones)

        R = 5

        def agg_loop(g_ref):
            def wait_g(k, b):
                pltpu.make_async_copy(g_ref.at[srcb.at[k]], rows[b], sg[b]).wait()

            def wait_s(k, b):
                pltpu.make_async_copy(
                    rows[b], agg_sh.at[dstb.at[k]], ss[b]).wait()

            pltpu.async_copy(g_ref.at[srcb.at[0]], rows[0], sg[0])
            pltpu.async_copy(g_ref.at[srcb.at[1]], rows[1], sg[1])

            def a_body(i, carry):
                for j in range(R):
                    k = R * i + j
                    bn = (j + 2) % R

                    @pl.when(k >= 3)
                    def _():
                        wait_s(jnp.maximum(k - 3, 0), bn)

                    @pl.when(k + 2 < NK)
                    def _():
                        pltpu.async_copy(
                            g_ref.at[srcb.at[k + 2]], rows[bn], sg[bn])
                    wait_g(k, j)
                    pltpu.async_copy(
                        rows[j], agg_sh.at[dstb.at[k]], ss[j], add=True)
                    hist_chunk(k)
                return carry
            lax.fori_loop(0, NK // R, a_body, 0)
            for k in range(NK - 3, NK):
                wait_s(k, k % R)

        @pl.when(c == 0)
        def _():
            agg_loop(ga_h)

        @pl.when(c == 1)
        def _():
            agg_loop(gb_h)

        if want_deg:
            @pl.when(c == 0)
            def _():
                pltpu.sync_copy(hist, dparts_sh.at[s])
        plsc.subcore_barrier()

        for q in range(RPT // FL):
            pltpu.sync_copy(agg_sh.at[pl.ds(s * RPT + q * FL, FL)], zstg)
            pltpu.sync_copy(zstg, part_h.at[c, pl.ds(s * RPT + q * FL, FL)])

        if want_deg:
            @pl.when(c == 0)
            def _():
                def r_body(i, carry):
                    tmp[pl.ds(i * 16, 16)] = jnp.zeros((16,), jnp.float32)
                    return carry
                lax.fori_loop(0, SEG // 16, r_body, 0)
                for t in range(NS):
                    pltpu.sync_copy(
                        dparts_sh.at[t, pl.ds(s * SEG, SEG)],
                        hist.at[pl.ds(0, SEG)])

                    def add_body(i, carry):
                        sl = pl.ds(i * 16, 16)
                        tmp[sl] = tmp[sl] + hist[sl]
                        return carry
                    lax.fori_loop(0, SEG // 16, add_body, 0)
                pltpu.sync_copy(tmp, deg_h.at[pl.ds(s * SEG, SEG)])

    ot = tuple(out_type) if want_deg else out_type[0]
    return pl.kernel(
        body, out_type=ot, mesh=mesh, scratch_types=scratch,
        compiler_params=pltpu.CompilerParams(
            use_tc_tiling_on_sc=False, needs_layout_passes=False),
        name="sc_seg_agg")


def _tc_in(x, Wn, Ws, b):
    N = x.shape[0]
    BN = 1000
    grid = (N // BN,)

    def body(xb, wn, ws, bb, g_out, s_out):
        xv = xb[...]
        g_out[...] = jnp.dot(xv, wn[...], preferred_element_type=jnp.float32)
        s_out[...] = jnp.dot(xv, ws[...], preferred_element_type=jnp.float32) + bb[...]

    return pl.pallas_call(
        body,
        grid=grid,
        in_specs=[
            pl.BlockSpec((BN, D), lambda i: (i, 0)),
            pl.BlockSpec((D, D), lambda i: (0, 0)),
            pl.BlockSpec((D, D), lambda i: (0, 0)),
            pl.BlockSpec((1, D), lambda i: (0, 0)),
        ],
        out_specs=[
            pl.BlockSpec((BN, D), lambda i: (i, 0)),
            pl.BlockSpec((BN, D), lambda i: (i, 0)),
        ],
        out_shape=[
            jax.ShapeDtypeStruct((N, D), jnp.float32),
            jax.ShapeDtypeStruct((N, D), jnp.float32),
        ],
        name="tc_in",
    )(x, Wn, Ws, b.reshape(1, D))


def _tc_mid(S, part, deg, Wn, Ws, b):
    N = S.shape[0]
    BN = 1000
    grid = (N // BN,)

    def body(sb, pa, pb, dg, wn, ws, bb, g_out, s_out):
        d = jnp.maximum(dg[...], 1.0)
        agg = jnp.concatenate([pa[0], pb[0]], axis=1)
        h = jnp.maximum(sb[...] + agg / d, 0.0)
        g_out[...] = jnp.dot(h, wn[...], preferred_element_type=jnp.float32)
        s_out[...] = jnp.dot(h, ws[...], preferred_element_type=jnp.float32) + bb[...]

    return pl.pallas_call(
        body,
        grid=grid,
        in_specs=[
            pl.BlockSpec((BN, D), lambda i: (i, 0)),
            pl.BlockSpec((1, BN, DH), lambda i: (0, i, 0)),
            pl.BlockSpec((1, BN, DH), lambda i: (1, i, 0)),
            pl.BlockSpec((BN, 1), lambda i: (i, 0)),
            pl.BlockSpec((D, D), lambda i: (0, 0)),
            pl.BlockSpec((D, D), lambda i: (0, 0)),
            pl.BlockSpec((1, D), lambda i: (0, 0)),
        ],
        out_specs=[
            pl.BlockSpec((BN, D), lambda i: (i, 0)),
            pl.BlockSpec((BN, D), lambda i: (i, 0)),
        ],
        out_shape=[
            jax.ShapeDtypeStruct((N, D), jnp.float32),
            jax.ShapeDtypeStruct((N, D), jnp.float32),
        ],
        name="tc_mid",
    )(S, part, part, deg, Wn, Ws, b.reshape(1, D))


def _tc_out(S, part, deg, Wp1, bp1, Wp2, bp2):
    N = S.shape[0]
    P_HID = Wp1.shape[1]
    P_OUT = Wp2.shape[1]
    BN = 1000
    grid = (N // BN,)

    def body(sb, pa, pb, dg, w1, b1b, w2, b2b, out):
        d = jnp.maximum(dg[...], 1.0)
        agg = jnp.concatenate([pa[0], pb[0]], axis=1)
        h = jnp.maximum(sb[...] + agg / d, 0.0)
        t = jnp.maximum(
            jnp.dot(h, w1[...], preferred_element_type=jnp.float32) + b1b[...], 0.0)
        out[...] = jnp.dot(t, w2[...], preferred_element_type=jnp.float32) + b2b[...]

    return pl.pallas_call(
        body,
        grid=grid,
        in_specs=[
            pl.BlockSpec((BN, D), lambda i: (i, 0)),
            pl.BlockSpec((1, BN, DH), lambda i: (0, i, 0)),
            pl.BlockSpec((1, BN, DH), lambda i: (1, i, 0)),
            pl.BlockSpec((BN, 1), lambda i: (i, 0)),
            pl.BlockSpec((D, P_HID), lambda i: (0, 0)),
            pl.BlockSpec((1, P_HID), lambda i: (0, 0)),
            pl.BlockSpec((P_HID, P_OUT), lambda i: (0, 0)),
            pl.BlockSpec((1, P_OUT), lambda i: (0, 0)),
        ],
        out_specs=pl.BlockSpec((BN, P_OUT), lambda i: (i, 0)),
        out_shape=jax.ShapeDtypeStruct((N, P_OUT), jnp.float32),
        name="tc_out",
    )(S, part, part, deg, Wp1, bp1.reshape(1, P_HID), Wp2, bp2.reshape(1, P_OUT))


def kernel(x, edge_index, W_self1, W_nbr1, b1, W_self2, W_nbr2, b2,
           Wp1, bp1, Wp2, bp2):
    N = x.shape[0]
    E = edge_index.shape[1]
    src2d = edge_index[0].reshape(E // CH, CH)
    dst2d = edge_index[1].reshape(E // CH, CH)
    zeros2d = jnp.zeros((128, DH), jnp.float32)

    agg_deg = _sc_agg(E, N, True)
    agg = _sc_agg(E, N, False)

    G1, S1 = _tc_in(x, W_nbr1, W_self1, b1)
    part1, deg, _ = agg_deg(src2d, dst2d, G1[:, :DH], G1[:, DH:], zeros2d)
    degc = deg[:N].reshape(N, 1)
    G2, S2 = _tc_mid(S1, part1, degc, W_nbr2, W_self2, b2)
    part2 = agg(src2d, dst2d, G2[:, :DH], G2[:, DH:], zeros2d)
    return _tc_out(S2, part2, degc, Wp1, bp1, Wp2, bp2)


# speedup vs baseline: 11.1347x; 1.0018x over previous
import jax
import jax.numpy as jnp
from jax import lax
from jax.experimental import pallas as pl
from jax.experimental.pallas import tpu as pltpu
from jax.experimental.pallas import tpu_sc as plsc

D = 128
DH = D // 2
NC = 2
NS = 16
CH = 80
R = 5


def _sc_agg(E, N, want_deg):
    NB = E // CH
    NK = NB // NS
    NP = ((N + 255) // 256) * 256
    RPT = NP // NS
    SEG = NP // NS
    FL = 128
    mesh = plsc.VectorSubcoreMesh(core_axis_name="c", subcore_axis_name="s")

    out_type = [jax.ShapeDtypeStruct((NC, NP, DH), jnp.float32)]
    scratch = (
        [pltpu.VMEM((NK, CH), jnp.int32)] * 2
        + [pltpu.VMEM((CH, DH), jnp.float32)] * R
        + [pltpu.VMEM((FL, DH), jnp.float32)]
        + [pltpu.VMEM_SHARED((NP, DH), jnp.float32)]
        + [pltpu.SemaphoreType.DMA] * (2 * R)
    )
    if want_deg:
        out_type.append(jax.ShapeDtypeStruct((NP,), jnp.float32))
        out_type.append(jax.ShapeDtypeStruct((NS, NP), jnp.float32))
        scratch += [
            pltpu.VMEM((NP,), jnp.float32),
            pltpu.VMEM((SEG,), jnp.float32),
        ]

    def body(src2d_h, dst2d_h, ga_h, gb_h, z_h, *rest):
        if want_deg:
            part_h, deg_h, dparts_h = rest[:3]
            rest = rest[3:]
            hist, tmp = rest[4 + 3 * R:]
        else:
            part_h = rest[0]
            rest = rest[1:]
        srcb, dstb = rest[0], rest[1]
        rows = rest[2:2 + R]
        zstg = rest[2 + R]
        agg_sh = rest[3 + R]
        sg = rest[4 + R:4 + 2 * R]
        ss = rest[4 + 2 * R:4 + 3 * R]
        c = lax.axis_index("c")
        s = lax.axis_index("s")

        pltpu.sync_copy(z_h, zstg)
        for q in range(RPT // FL):
            pltpu.sync_copy(zstg, agg_sh.at[pl.ds(s * RPT + q * FL, FL)])
        if want_deg:
            def z_body(i, carry):
                hist[pl.ds(i * 16, 16)] = jnp.zeros((16,), jnp.float32)
                return carry
            lax.fori_loop(0, NP // 16, z_body, 0)
        plsc.subcore_barrier()

        pltpu.sync_copy(src2d_h.at[pl.ds(s * NK, NK)], srcb)
        pltpu.sync_copy(dst2d_h.at[pl.ds(s * NK, NK)], dstb)

        ones = jnp.ones((16,), jnp.float32)

        def hist_chunk(k):
            for j in range(CH // 16):
                v = dstb[k, pl.ds(j * 16, 16)]
                plsc.addupdate_scatter(hist, [v], ones)

        LD = 2
        LAG = R - LD

        def agg_loop(g_ref, with_hist):
            def wait_g(k, b):
                pltpu.make_async_copy(g_ref.at[srcb.at[k]], rows[b], sg[b]).wait()

            def wait_s(k, b):
                pltpu.make_async_copy(
                    rows[b], agg_sh.at[dstb.at[k]], ss[b]).wait()

            for k in range(LD):
                pltpu.async_copy(g_ref.at[srcb.at[k]], rows[k], sg[k])

            def a_body(i, carry):
                for j in range(R):
                    k = R * i + j
                    bn = (j + LD) % R

                    @pl.when(k >= LAG)
                    def _():
                        wait_s(jnp.maximum(k - LAG, 0), bn)

                    @pl.when(k + LD < NK)
                    def _():
                        pltpu.async_copy(
                            g_ref.at[srcb.at[k + LD]], rows[bn], sg[bn])
                    wait_g(k, j)
                    pltpu.async_copy(
                        rows[j], agg_sh.at[dstb.at[k]], ss[j], add=True)
                    if with_hist:
                        hist_chunk(k)
                return carry
            lax.fori_loop(0, NK // R, a_body, 0)
            for k in range(NK - LAG, NK):
                wait_s(k, k % R)

        @pl.when(c == 0)
        def _():
            agg_loop(ga_h, want_deg)
            if want_deg:
                pltpu.sync_copy(hist, dparts_h.at[s])

        @pl.when(c == 1)
        def _():
            agg_loop(gb_h, False)
        plsc.subcore_barrier()

        for q in range(RPT // FL):
            pltpu.sync_copy(agg_sh.at[pl.ds(s * RPT + q * FL, FL)], zstg)
            pltpu.sync_copy(zstg, part_h.at[c, pl.ds(s * RPT + q * FL, FL)])

        if want_deg:
            @pl.when(c == 0)
            def _():
                def r_body(i, carry):
                    tmp[pl.ds(i * 16, 16)] = jnp.zeros((16,), jnp.float32)
                    return carry
                lax.fori_loop(0, SEG // 16, r_body, 0)
                for t in range(NS):
                    pltpu.sync_copy(
                        dparts_h.at[t, pl.ds(s * SEG, SEG)],
                        hist.at[pl.ds(0, SEG)])

                    def add_body(i, carry):
                        sl = pl.ds(i * 16, 16)
                        tmp[sl] = tmp[sl] + hist[sl]
                        return carry
                    lax.fori_loop(0, SEG // 16, add_body, 0)
                pltpu.sync_copy(tmp, deg_h.at[pl.ds(s * SEG, SEG)])

    ot = tuple(out_type) if want_deg else out_type[0]
    return pl.kernel(
        body, out_type=ot, mesh=mesh, scratch_types=scratch,
        compiler_params=pltpu.CompilerParams(
            use_tc_tiling_on_sc=False, needs_layout_passes=False),
        name="sc_seg_agg")


def _tc_in(x, Wn, Ws, b):
    N = x.shape[0]
    BN = 1000
    grid = (N // BN,)

    def body(xb, wn, ws, bb, g_out, s_out):
        xv = xb[...]
        g_out[...] = jnp.dot(xv, wn[...], preferred_element_type=jnp.float32)
        s_out[...] = jnp.dot(xv, ws[...], preferred_element_type=jnp.float32) + bb[...]

    return pl.pallas_call(
        body,
        grid=grid,
        in_specs=[
            pl.BlockSpec((BN, D), lambda i: (i, 0)),
            pl.BlockSpec((D, D), lambda i: (0, 0)),
            pl.BlockSpec((D, D), lambda i: (0, 0)),
            pl.BlockSpec((1, D), lambda i: (0, 0)),
        ],
        out_specs=[
            pl.BlockSpec((BN, D), lambda i: (i, 0)),
            pl.BlockSpec((BN, D), lambda i: (i, 0)),
        ],
        out_shape=[
            jax.ShapeDtypeStruct((N, D), jnp.float32),
            jax.ShapeDtypeStruct((N, D), jnp.float32),
        ],
        name="tc_in",
    )(x, Wn, Ws, b.reshape(1, D))


def _tc_mid(S, part, deg, Wn, Ws, b):
    N = S.shape[0]
    BN = 1000
    grid = (N // BN,)

    def body(sb, pa, pb, dg, wn, ws, bb, g_out, s_out):
        d = jnp.maximum(dg[...], 1.0)
        agg = jnp.concatenate([pa[0], pb[0]], axis=1)
        h = jnp.maximum(sb[...] + agg / d, 0.0)
        g_out[...] = jnp.dot(h, wn[...], preferred_element_type=jnp.float32)
        s_out[...] = jnp.dot(h, ws[...], preferred_element_type=jnp.float32) + bb[...]

    return pl.pallas_call(
        body,
        grid=grid,
        in_specs=[
            pl.BlockSpec((BN, D), lambda i: (i, 0)),
            pl.BlockSpec((1, BN, DH), lambda i: (0, i, 0)),
            pl.BlockSpec((1, BN, DH), lambda i: (1, i, 0)),
            pl.BlockSpec((BN, 1), lambda i: (i, 0)),
            pl.BlockSpec((D, D), lambda i: (0, 0)),
            pl.BlockSpec((D, D), lambda i: (0, 0)),
            pl.BlockSpec((1, D), lambda i: (0, 0)),
        ],
        out_specs=[
            pl.BlockSpec((BN, D), lambda i: (i, 0)),
            pl.BlockSpec((BN, D), lambda i: (i, 0)),
        ],
        out_shape=[
            jax.ShapeDtypeStruct((N, D), jnp.float32),
            jax.ShapeDtypeStruct((N, D), jnp.float32),
        ],
        name="tc_mid",
    )(S, part, part, deg, Wn, Ws, b.reshape(1, D))


def _tc_out(S, part, deg, Wp1, bp1, Wp2, bp2):
    N = S.shape[0]
    P_HID = Wp1.shape[1]
    P_OUT = Wp2.shape[1]
    BN = 1000
    grid = (N // BN,)

    def body(sb, pa, pb, dg, w1, b1b, w2, b2b, out):
        d = jnp.maximum(dg[...], 1.0)
        agg = jnp.concatenate([pa[0], pb[0]], axis=1)
        h = jnp.maximum(sb[...] + agg / d, 0.0)
        t = jnp.maximum(
            jnp.dot(h, w1[...], preferred_element_type=jnp.float32) + b1b[...], 0.0)
        out[...] = jnp.dot(t, w2[...], preferred_element_type=jnp.float32) + b2b[...]

    return pl.pallas_call(
        body,
        grid=grid,
        in_specs=[
            pl.BlockSpec((BN, D), lambda i: (i, 0)),
            pl.BlockSpec((1, BN, DH), lambda i: (0, i, 0)),
            pl.BlockSpec((1, BN, DH), lambda i: (1, i, 0)),
            pl.BlockSpec((BN, 1), lambda i: (i, 0)),
            pl.BlockSpec((D, P_HID), lambda i: (0, 0)),
            pl.BlockSpec((1, P_HID), lambda i: (0, 0)),
            pl.BlockSpec((P_HID, P_OUT), lambda i: (0, 0)),
            pl.BlockSpec((1, P_OUT), lambda i: (0, 0)),
        ],
        out_specs=pl.BlockSpec((BN, P_OUT), lambda i: (i, 0)),
        out_shape=jax.ShapeDtypeStruct((N, P_OUT), jnp.float32),
        name="tc_out",
    )(S, part, part, deg, Wp1, bp1.reshape(1, P_HID), Wp2, bp2.reshape(1, P_OUT))


def kernel(x, edge_index, W_self1, W_nbr1, b1, W_self2, W_nbr2, b2,
           Wp1, bp1, Wp2, bp2):
    N = x.shape[0]
    E = edge_index.shape[1]
    src2d = edge_index[0].reshape(E // CH, CH)
    dst2d = edge_index[1].reshape(E // CH, CH)
    zeros2d = jnp.zeros((128, DH), jnp.float32)

    agg_deg = _sc_agg(E, N, True)
    agg = _sc_agg(E, N, False)

    G1, S1 = _tc_in(x, W_nbr1, W_self1, b1)
    part1, deg, _ = agg_deg(src2d, dst2d, G1[:, :DH], G1[:, DH:], zeros2d)
    degc = deg[:N].reshape(N, 1)
    G2, S2 = _tc_mid(S1, part1, degc, W_nbr2, W_self2, b2)
    part2 = agg(src2d, dst2d, G2[:, :DH], G2[:, DH:], zeros2d)
    return _tc_out(S2, part2, degc, Wp1, bp1, Wp2, bp2)
